# Initial kernel scaffold; baseline (speedup 1.0000x reference)
import jax
import jax.numpy as jnp
from jax.experimental import pallas as pl


def kernel(per_modality_input_tokens, timestamps, channel_embed, patch_size, input_res):
    raise NotImplementedError("write your pallas kernel here")


# speedup vs baseline: 3.5430x; 3.5430x over previous
import math

import jax
import jax.numpy as jnp
from jax import lax
from jax.experimental import pallas as pl
from jax.experimental.pallas import tpu as pltpu

BASE_GSD = 10.0
HW_BLK = 64


def _tc_body(gsd_ref, months_ref, ch_ref, x_ref, o_ref):
    f32 = jnp.float32
    gsd = gsd_ref[0, 0]
    hwb = pl.program_id(1)
    t, cg = months_ref.shape[1], ch_ref.shape[0]
    tcg = t * cg

    t_col = lax.broadcasted_iota(jnp.int32, (t, 16), 0).astype(f32)
    om16 = 1.0 / (10000.0 ** (
        lax.broadcasted_iota(jnp.int32, (t, 16), 1).astype(f32) / 16.0))
    arg = t_col * om16
    pos32 = jnp.concatenate([jnp.sin(arg), jnp.cos(arg)], axis=1)

    ang = months_ref[0].astype(f32) / f32(12.0 / (2.0 * math.pi))
    mon32 = jnp.concatenate([
        jnp.broadcast_to(jnp.sin(ang), (t, 16)),
        jnp.broadcast_to(jnp.cos(ang), (t, 16)),
    ], axis=1)

    ch_e = jnp.broadcast_to(ch_ref[...][None], (t, cg, 32)).reshape(tcg, 32)
    pos_e = jnp.broadcast_to(pos32[:, None, :], (t, cg, 32)).reshape(tcg, 32)
    mon_e = jnp.broadcast_to(mon32[:, None, :], (t, cg, 32)).reshape(tcg, 32)
    a1 = jnp.concatenate(
        [ch_e, pos_e, mon_e, jnp.zeros((tcg, 32), f32)], axis=1)

    hw = hwb * HW_BLK + lax.broadcasted_iota(jnp.int32, (HW_BLK, 8), 0)
    iv = (hw // 16).astype(f32) * gsd
    jv = (hw % 16).astype(f32) * gsd
    om8 = 1.0 / (10000.0 ** (
        lax.broadcasted_iota(jnp.int32, (HW_BLK, 8), 1).astype(f32) / 8.0))
    aj = jv * om8
    ai = iv * om8
    sp = jnp.concatenate([
        jnp.zeros((HW_BLK, 96), f32),
        jnp.sin(aj), jnp.cos(aj), jnp.sin(ai), jnp.cos(ai),
    ], axis=1)

    x = x_ref[0]
    o_ref[0] = x + a1[None, :, :] + sp[:, None, :]


def kernel(per_modality_input_tokens, timestamps, channel_embed, patch_size,
           input_res):
    x = per_modality_input_tokens
    b, h, w, t, cg, D = x.shape
    xr = x.reshape(b, h * w, t * cg, D)
    months = timestamps[:, 1, :].astype(jnp.int32).reshape(b, t, 1)
    gsd = (jnp.asarray(input_res).astype(jnp.float32)
           * jnp.asarray(patch_size).astype(jnp.float32) / BASE_GSD)
    gsd = gsd.reshape(1, 1)
    out = pl.pallas_call(
        _tc_body,
        grid=(b, (h * w) // HW_BLK),
        in_specs=[
            pl.BlockSpec(memory_space=pltpu.SMEM),
            pl.BlockSpec((1, t, 1), lambda bi, hi: (bi, 0, 0)),
            pl.BlockSpec((cg, 32), lambda bi, hi: (0, 0)),
            pl.BlockSpec((1, HW_BLK, t * cg, D), lambda bi, hi: (bi, hi, 0, 0)),
        ],
        out_specs=pl.BlockSpec((1, HW_BLK, t * cg, D),
                               lambda bi, hi: (bi, hi, 0, 0)),
        out_shape=jax.ShapeDtypeStruct(xr.shape, xr.dtype),
        compiler_params=pltpu.CompilerParams(
            dimension_semantics=("parallel", "parallel")),
    )(gsd, months, channel_embed, xr)
    return out.reshape(b, h, w, t, cg, D)
